# baseline (speedup 1.0000x reference)
import functools

import jax
import jax.numpy as jnp
from jax import lax
from jax.experimental import pallas as pl
from jax.experimental.pallas import tpu as pltpu
from jax.experimental.pallas import tpu_sc as plsc

N_NODES = 100000
N_FEATS = 9
VOCAB = 100
HIDDEN = 512

NC = 2
NS = 16
NW = NC * NS
N_HSPLIT = 16
N_GSPLIT = NW // N_HSPLIT
HSLICE = HIDDEN // N_HSPLIT
HPAIR = HSLICE // 2
ROWS = N_FEATS * VOCAB
G_NODES = N_NODES // N_GSPLIT
CHUNK = 400
N_CHUNKS = G_NODES // CHUNK
TILES = CHUNK // 16


def _sc_body(idx_hbm, tab_hbm, out_hbm, table_v, idx_v, stage_v):
    c = lax.axis_index("c")
    s = lax.axis_index("s")
    wid = s * NC + c
    hid = wid % N_HSPLIT
    ng = wid // N_HSPLIT

    pltpu.sync_copy(tab_hbm.at[hid], table_v)

    lane = lax.iota(jnp.int32, 16)

    def chunk_body(k, _):
        gbase = ng * G_NODES + k * CHUNK
        for f in range(N_FEATS):
            pltpu.sync_copy(
                idx_hbm.at[pl.ds(f * N_NODES + gbase, CHUNK)],
                idx_v.at[f])

        def tile_body(t, _):
            nb = t * 16
            vis = [idx_v[f, pl.ds(nb, 16)] * HPAIR for f in range(N_FEATS)]
            rows = lane + nb
            for j in range(HPAIR):
                cj = jnp.full((16,), j, jnp.int32)
                g = plsc.bitcast(
                    plsc.load_gather(table_v, [vis[0] + cj]), jnp.bfloat16)
                for f in range(1, N_FEATS):
                    g = g + plsc.bitcast(
                        plsc.load_gather(table_v, [vis[f] + cj]), jnp.bfloat16)
                lo, hi = plsc.unpack(g, format=plsc.PackFormat.INTERLEAVED,
                                     preferred_element_type=jnp.float32)
                plsc.store_scatter(
                    stage_v, [rows, jnp.full((16,), 2 * j, jnp.int32)], lo)
                plsc.store_scatter(
                    stage_v, [rows, jnp.full((16,), 2 * j + 1, jnp.int32)], hi)
            return 0

        lax.fori_loop(0, TILES, tile_body, 0)
        pltpu.sync_copy(
            stage_v,
            out_hbm.at[pl.ds(gbase, CHUNK), pl.ds(hid * HSLICE, HSLICE)])
        return 0

    lax.fori_loop(0, N_CHUNKS, chunk_body, 0)


@jax.jit
def kernel(x, tables):
    offs = (jnp.arange(N_FEATS, dtype=jnp.int32) * VOCAB)[None, :]
    idx_t = (x.astype(jnp.int32) + offs).T.reshape(-1)

    tb = tables.reshape(ROWS, HIDDEN).astype(jnp.bfloat16)
    tb = tb.reshape(ROWS, N_HSPLIT, HPAIR, 2).transpose(1, 0, 2, 3)
    tb_packed = lax.bitcast_convert_type(tb, jnp.int32).reshape(
        N_HSPLIT, ROWS * HPAIR)

    mesh = plsc.VectorSubcoreMesh(
        core_axis_name="c", subcore_axis_name="s",
        num_cores=NC, num_subcores=NS)
    f = pl.kernel(
        _sc_body,
        out_type=jax.ShapeDtypeStruct((N_NODES, HIDDEN), jnp.float32),
        mesh=mesh,
        scratch_types=[
            pltpu.VMEM((ROWS * HPAIR,), jnp.int32),
            pltpu.VMEM((N_FEATS, CHUNK), jnp.int32),
            pltpu.VMEM((CHUNK, HSLICE), jnp.float32),
        ],
        compiler_params=pltpu.CompilerParams(
            use_tc_tiling_on_sc=False, needs_layout_passes=False),
    )
    return f(idx_t, tb_packed)


# speedup vs baseline: 1.6835x; 1.6835x over previous
import functools

import jax
import jax.numpy as jnp
from jax import lax
from jax.experimental import pallas as pl
from jax.experimental.pallas import tpu as pltpu
from jax.experimental.pallas import tpu_sc as plsc

N_NODES = 100000
N_FEATS = 9
VOCAB = 100
HIDDEN = 512

NC = 2
NS = 16
NW = NC * NS
N_HSPLIT = 16
N_GSPLIT = NW // N_HSPLIT
HSLICE = HIDDEN // N_HSPLIT
HPAIR = HSLICE // 2
ROWS = N_FEATS * VOCAB
G_NODES = N_NODES // N_GSPLIT
CHUNK = 400
N_CHUNKS = G_NODES // CHUNK
TILES = CHUNK // 16


def _sc_body(idx_hbm, tab_hbm, out_hbm, table_v, idx_v, stage_v):
    c = lax.axis_index("c")
    s = lax.axis_index("s")
    wid = s * NC + c
    hid = wid % N_HSPLIT
    ng = wid // N_HSPLIT

    pltpu.sync_copy(tab_hbm.at[hid], table_v)

    lane = lax.iota(jnp.int32, 16)

    def chunk_body(k, _):
        gbase = ng * G_NODES + k * CHUNK
        for f in range(N_FEATS):
            pltpu.sync_copy(
                idx_hbm.at[pl.ds(f * N_NODES + gbase, CHUNK)],
                idx_v.at[f])

        def tile_body(t, _):
            nb = t * 16
            vis = [idx_v[f, pl.ds(nb, 16)] for f in range(N_FEATS)]
            rows = lane + nb
            for j in range(HPAIR):
                cj = jnp.full((16,), j * ROWS, jnp.int32)
                g = plsc.bitcast(
                    plsc.load_gather(table_v, [vis[0] + cj]), jnp.bfloat16)
                for f in range(1, N_FEATS):
                    g = g + plsc.bitcast(
                        plsc.load_gather(table_v, [vis[f] + cj]), jnp.bfloat16)
                lo, hi = plsc.unpack(g, format=plsc.PackFormat.INTERLEAVED,
                                     preferred_element_type=jnp.float32)
                plsc.store_scatter(
                    stage_v, [rows, jnp.full((16,), 2 * j, jnp.int32)], lo)
                plsc.store_scatter(
                    stage_v, [rows, jnp.full((16,), 2 * j + 1, jnp.int32)], hi)
            return 0

        lax.fori_loop(0, TILES, tile_body, 0)
        pltpu.sync_copy(
            stage_v,
            out_hbm.at[pl.ds(gbase, CHUNK), pl.ds(hid * HSLICE, HSLICE)])
        return 0

    lax.fori_loop(0, N_CHUNKS, chunk_body, 0)


@jax.jit
def kernel(x, tables):
    offs = (jnp.arange(N_FEATS, dtype=jnp.int32) * VOCAB)[None, :]
    idx_t = (x.astype(jnp.int32) + offs).T.reshape(-1)

    tb = tables.reshape(ROWS, HIDDEN).astype(jnp.bfloat16)
    tb = tb.reshape(ROWS, N_HSPLIT, HPAIR, 2).transpose(1, 0, 2, 3)
    tb_packed = lax.bitcast_convert_type(tb, jnp.int32).transpose(
        0, 2, 1).reshape(N_HSPLIT, ROWS * HPAIR)

    mesh = plsc.VectorSubcoreMesh(
        core_axis_name="c", subcore_axis_name="s",
        num_cores=NC, num_subcores=NS)
    f = pl.kernel(
        _sc_body,
        out_type=jax.ShapeDtypeStruct((N_NODES, HIDDEN), jnp.float32),
        mesh=mesh,
        scratch_types=[
            pltpu.VMEM((ROWS * HPAIR,), jnp.int32),
            pltpu.VMEM((N_FEATS, CHUNK), jnp.int32),
            pltpu.VMEM((CHUNK, HSLICE), jnp.float32),
        ],
        compiler_params=pltpu.CompilerParams(
            use_tc_tiling_on_sc=False, needs_layout_passes=False),
    )
    return f(idx_t, tb_packed)


# speedup vs baseline: 2.1513x; 1.2779x over previous
import functools

import jax
import jax.numpy as jnp
from jax import lax
from jax.experimental import pallas as pl
from jax.experimental.pallas import tpu as pltpu
from jax.experimental.pallas import tpu_sc as plsc

N_NODES = 100000
N_FEATS = 9
VOCAB = 100
HIDDEN = 512

NC = 2
NS = 16
NW = NC * NS
N_HSPLIT = 16
N_GSPLIT = NW // N_HSPLIT
HSLICE = HIDDEN // N_HSPLIT
HPAIR = HSLICE // 2
ROWS = N_FEATS * VOCAB
G_NODES = N_NODES // N_GSPLIT
CHUNK = 400
N_CHUNKS = G_NODES // CHUNK
TILES = CHUNK // 16


def _sc_body(idx_hbm, tab_hbm, out_hbm, table_sp, idx_v, gblk, stage_v, sems):
    c = lax.axis_index("c")
    s = lax.axis_index("s")
    wid = s * NC + c
    hid = wid % N_HSPLIT
    ng = wid // N_HSPLIT

    @pl.when(s == 0)
    def _():
        pltpu.sync_copy(tab_hbm, table_sp)

    plsc.subcore_barrier()

    def start_gathers(t, b):
        nb = t * 16
        for f in range(N_FEATS):
            vis = idx_v[f, pl.ds(nb, 16)]
            pltpu.async_copy(table_sp.at[hid].at[vis], gblk.at[b, f],
                             sems.at[b])

    def wait_gathers(t, b):
        nb = t * 16
        for f in range(N_FEATS):
            vis = idx_v[f, pl.ds(nb, 16)]
            pltpu.make_async_copy(
                table_sp.at[hid].at[vis], gblk.at[b, f], sems.at[b]).wait()

    def compute_tile(t, b):
        nb = t * 16
        for l in range(16):
            g = plsc.bitcast(gblk[b, 0, l], jnp.bfloat16)
            for f in range(1, N_FEATS):
                g = g + plsc.bitcast(gblk[b, f, l], jnp.bfloat16)
            lo, hi = plsc.unpack(g, format=plsc.PackFormat.INTERLEAVED,
                                 preferred_element_type=jnp.float32)
            stage_v[nb + l, pl.ds(0, 16)] = lo
            stage_v[nb + l, pl.ds(16, 16)] = hi

    def chunk_body(k, _):
        gbase = ng * G_NODES + k * CHUNK
        for f in range(N_FEATS):
            pltpu.sync_copy(
                idx_hbm.at[pl.ds(f * N_NODES + gbase, CHUNK)],
                idx_v.at[f])

        start_gathers(0, 0)

        def tile_body(i, _):
            b = lax.rem(i, 2)

            @pl.when(i < TILES - 1)
            def _():
                start_gathers(i + 1, 1 - b)

            wait_gathers(i, b)
            compute_tile(i, b)
            return 0

        lax.fori_loop(0, TILES, tile_body, 0)
        pltpu.sync_copy(
            stage_v,
            out_hbm.at[pl.ds(gbase, CHUNK), pl.ds(hid * HSLICE, HSLICE)])
        return 0

    lax.fori_loop(0, N_CHUNKS, chunk_body, 0)


@jax.jit
def kernel(x, tables):
    offs = (jnp.arange(N_FEATS, dtype=jnp.int32) * VOCAB)[None, :]
    idx_t = (x.astype(jnp.int32) + offs).T.reshape(-1)

    tb = tables.reshape(ROWS, HIDDEN).astype(jnp.bfloat16)
    tb = tb.reshape(ROWS, N_HSPLIT, HPAIR, 2).transpose(1, 0, 2, 3)
    tb_packed = lax.bitcast_convert_type(tb, jnp.int32)

    mesh = plsc.VectorSubcoreMesh(
        core_axis_name="c", subcore_axis_name="s",
        num_cores=NC, num_subcores=NS)
    f = pl.kernel(
        _sc_body,
        out_type=jax.ShapeDtypeStruct((N_NODES, HIDDEN), jnp.float32),
        mesh=mesh,
        scratch_types=[
            pltpu.VMEM_SHARED((N_HSPLIT, ROWS, HPAIR), jnp.int32),
            pltpu.VMEM((N_FEATS, CHUNK), jnp.int32),
            pltpu.VMEM((2, N_FEATS, 16, HPAIR), jnp.int32),
            pltpu.VMEM((CHUNK, HSLICE), jnp.float32),
            pltpu.SemaphoreType.DMA((2,)),
        ],
        compiler_params=pltpu.CompilerParams(
            use_tc_tiling_on_sc=False, needs_layout_passes=False),
    )
    return f(idx_t, tb_packed)
